# baseline (speedup 1.0000x reference)
import functools

import jax
import jax.numpy as jnp
import numpy as np
from jax.experimental import pallas as pl
from jax.experimental.pallas import tpu as pltpu

DT = 0.01
DH = 0.1
R2 = 1.5 * 1.5
COORDS_WEIGHT = 1.0
NP = 32


def _silu(x):
    return x * jax.nn.sigmoid(x)


def _enflow_kernel(n_layers, bb, ne,
                   h_ref, pos_ref, vel_ref, g_ref,
                   r_ref, t_ref, rt_ref, vmask_ref,
                   wa_ref, wb_ref, wr_ref, be1_ref,
                   we2_ref, be2_ref, wc1_ref, bc1_ref, wc2_ref,
                   wn1h_ref, wn1a_ref, bn1_ref, wn2_ref, bn2_ref,
                   ws_ref, bs_ref,
                   h_out, pos_out, vel_out, g_out, s_out):
    nf = h_ref.shape[-1]
    h = h_ref[...]
    pos = pos_ref[...]
    vel = vel_ref[...]
    g = g_ref[...]

    rm = r_ref[...]
    tm = t_ref[...]
    rtm = rt_ref[...]
    vmask = vmask_ref[...]

    dot = functools.partial(jnp.dot, preferred_element_type=jnp.float32)
    hdot = functools.partial(jnp.dot, preferred_element_type=jnp.float32,
                             precision=jax.lax.Precision.HIGHEST)

    s_acc = jnp.zeros((bb * NP, 1), jnp.float32)

    for l in range(n_layers):
        pos_i = jnp.concatenate(
            [hdot(rm, pos[b]) for b in range(bb)], axis=0)
        pos_j = jnp.concatenate(
            [hdot(tm, pos[b]) for b in range(bb)], axis=0)
        diff = pos_i - pos_j
        radial = jnp.sum(diff * diff, axis=-1, keepdims=True)
        maskf = jnp.where(radial < R2, vmask, 0.0)

        hf2 = h.reshape(bb * NP, nf)
        a = dot(hf2, wa_ref[l]) + be1_ref[l]
        b2 = dot(hf2, wb_ref[l])
        a3 = a.reshape(bb, NP, nf)
        b3 = b2.reshape(bb, NP, nf)
        pre = jnp.concatenate(
            [hdot(rm, a3[b]) + hdot(tm, b3[b]) for b in range(bb)],
            axis=0)
        pre = pre + radial * wr_ref[l]
        m = _silu(pre)
        m = _silu(dot(m, we2_ref[l]) + be2_ref[l])
        m = m * maskf
        c1 = _silu(dot(m, wc1_ref[l]) + bc1_ref[l])
        cm = dot(c1, wc2_ref[l]) * maskf
        trans = diff * cm

        m3 = m.reshape(bb, ne, nf)
        t3 = trans.reshape(bb, ne, 3)
        agg = jnp.concatenate(
            [hdot(rtm, m3[b]) for b in range(bb)], axis=0)
        force = jnp.concatenate(
            [hdot(rtm, t3[b]) for b in range(bb)],
            axis=0).reshape(bb, NP, 3) * COORDS_WEIGHT

        hn = _silu(dot(hf2, wn1h_ref[l]) + dot(agg, wn1a_ref[l]) + bn1_ref[l])
        hforce = dot(hn, wn2_ref[l]) + bn2_ref[l]
        s = dot(agg, ws_ref[l]) + bs_ref[l]

        s3 = s.reshape(bb, NP, 1)
        vel = jnp.exp(s3) * vel + force * DT
        pos = pos + vel * DT
        g = g + hforce.reshape(bb, NP, nf) * DH
        h = h + g * DH
        s_acc = s_acc + s

    h_out[...] = h
    pos_out[...] = pos
    vel_out[...] = vel
    g_out[...] = g
    s_out[...] = s_acc.reshape(bb, NP, 1)


def kernel(h, pos, vel, g, params):
    B, N, nf = h.shape
    n_layers = len(params)
    bb = 4
    ne = ((N * N + 7) // 8) * 8
    pad = NP - N

    hp = jnp.pad(h, ((0, 0), (0, pad), (0, 0)))
    posp = jnp.pad(pos, ((0, 0), (0, pad), (0, 0)))
    velp = jnp.pad(vel, ((0, 0), (0, pad), (0, 0)))
    gp = jnp.pad(g, ((0, 0), (0, pad), (0, 0)))

    e = np.arange(ne)
    i_of_e = np.minimum(e // N, N - 1)
    j_of_e = np.minimum(e % N, N - 1)
    valid = (e < N * N) & (i_of_e != j_of_e)
    r_np = np.zeros((ne, NP), np.float32)
    t_np = np.zeros((ne, NP), np.float32)
    r_np[e[valid], i_of_e[valid]] = 1.0
    t_np[e[valid], j_of_e[valid]] = 1.0
    rt_np = r_np.T.copy()
    vmask_np = np.tile(valid.astype(np.float32)[:, None], (bb, 1)).reshape(
        bb * ne, 1)
    r_m = jnp.asarray(r_np)
    t_m = jnp.asarray(t_np)
    rt_m = jnp.asarray(rt_np)
    vmask = jnp.asarray(vmask_np)

    st = lambda name: jnp.stack([p[name] for p in params])
    we1 = st("We1")
    wa = we1[:, :nf]
    wb = we1[:, nf:2 * nf]
    wr = we1[:, 2 * nf:]
    be1 = st("be1")[:, None, :]
    we2 = st("We2")
    be2 = st("be2")[:, None, :]
    wc1 = st("Wc1")
    bc1 = st("bc1")[:, None, :]
    wc2 = st("Wc2")
    wn1 = st("Wn1")
    wn1h = wn1[:, :nf]
    wn1a = wn1[:, nf:]
    bn1 = st("bn1")[:, None, :]
    wn2 = st("Wn2")
    bn2 = st("bn2")[:, None, :]
    ws = st("Ws")
    bs = st("bs")[:, :, None]

    def wspec(x):
        return pl.BlockSpec(x.shape, lambda i: (0,) * x.ndim)

    def bspec(last):
        return pl.BlockSpec((bb, NP, last), lambda i: (i, 0, 0))

    consts = (r_m, t_m, rt_m, vmask)
    weights = (wa, wb, wr, be1, we2, be2, wc1, bc1, wc2,
               wn1h, wn1a, bn1, wn2, bn2, ws, bs)

    outs = pl.pallas_call(
        functools.partial(_enflow_kernel, n_layers, bb, ne),
        grid=(B // bb,),
        in_specs=[bspec(nf), bspec(3), bspec(3), bspec(nf)]
                 + [wspec(w) for w in consts + weights],
        out_specs=[bspec(nf), bspec(3), bspec(3), bspec(nf), bspec(1)],
        out_shape=[
            jax.ShapeDtypeStruct((B, NP, nf), jnp.float32),
            jax.ShapeDtypeStruct((B, NP, 3), jnp.float32),
            jax.ShapeDtypeStruct((B, NP, 3), jnp.float32),
            jax.ShapeDtypeStruct((B, NP, nf), jnp.float32),
            jax.ShapeDtypeStruct((B, NP, 1), jnp.float32),
        ],
        compiler_params=pltpu.CompilerParams(
            dimension_semantics=("parallel",)),
    )(hp, posp, velp, gp, *consts, *weights)

    h_o, pos_o, vel_o, g_o, s_o = outs
    ldj = jnp.sum(s_o[:, :N])
    return (h_o[:, :N], pos_o[:, :N], vel_o[:, :N], g_o[:, :N], ldj)


# speedup vs baseline: 4.4117x; 4.4117x over previous
import functools

import jax
import jax.numpy as jnp
import numpy as np
from jax.experimental import pallas as pl
from jax.experimental.pallas import tpu as pltpu

DT = 0.01
DH = 0.1
R2 = 1.5 * 1.5
COORDS_WEIGHT = 1.0
NP = 32


def _silu(x):
    return x * jax.nn.sigmoid(x)


def _enflow_kernel(n_layers, n_atoms, bb,
                   h_ref, pos_ref, vel_ref, g_ref, bmask_ref,
                   wa_ref, wb_ref, wr_ref, be1_ref,
                   we2_ref, be2_ref, wc1_ref, bc1_ref, wc2_ref,
                   wn1h_ref, wn1a_ref, bn1_ref, wn2_ref, bn2_ref,
                   ws_ref, bs_ref,
                   h_out, pos_out, vel_out, g_out, s_out):
    nf = h_ref.shape[-1]
    ni = n_atoms
    h = h_ref[...]
    pos = pos_ref[...]
    vel = vel_ref[...]
    g = g_ref[...]
    bmask = bmask_ref[...]

    dot = functools.partial(jnp.dot, preferred_element_type=jnp.float32)

    s_acc = jnp.zeros((bb * NP, 1), jnp.float32)

    for l in range(n_layers):
        diff = pos[:, :ni, None, :] - pos[:, None, :, :]
        radial = jnp.sum(diff * diff, axis=-1, keepdims=True)
        maskf = jnp.where(radial < R2, bmask, 0.0)

        hf2 = h.reshape(bb * NP, nf)
        a = dot(hf2, wa_ref[l]) + be1_ref[l]
        b2 = dot(hf2, wb_ref[l])
        a4 = a.reshape(bb, NP, nf)[:, :ni, None, :]
        b4 = b2.reshape(bb, NP, nf)[:, None, :, :]
        pre = a4 + b4 + radial * wr_ref[l]
        m = _silu(pre.reshape(bb * ni * NP, nf))
        m = _silu(dot(m, we2_ref[l]) + be2_ref[l])
        m4 = m.reshape(bb, ni, NP, nf) * maskf
        agg = jnp.sum(m4, axis=2)
        mflat = m4.reshape(bb * ni * NP, nf)
        c1 = _silu(dot(mflat, wc1_ref[l]) + bc1_ref[l])
        cm = dot(c1, wc2_ref[l])
        cm4 = cm.reshape(bb, ni, NP, 1) * maskf
        force = jnp.sum(diff * cm4, axis=2) * COORDS_WEIGHT

        aggp = jnp.pad(agg, ((0, 0), (0, NP - ni), (0, 0)))
        forcep = jnp.pad(force, ((0, 0), (0, NP - ni), (0, 0)))
        aggf = aggp.reshape(bb * NP, nf)
        hn = _silu(dot(hf2, wn1h_ref[l]) + dot(aggf, wn1a_ref[l]) + bn1_ref[l])
        hforce = dot(hn, wn2_ref[l]) + bn2_ref[l]
        s = dot(aggf, ws_ref[l]) + bs_ref[l]

        s3 = s.reshape(bb, NP, 1)
        vel = jnp.exp(s3) * vel + forcep * DT
        pos = pos + vel * DT
        g = g + hforce.reshape(bb, NP, nf) * DH
        h = h + g * DH
        s_acc = s_acc + s

    h_out[...] = h
    pos_out[...] = pos
    vel_out[...] = vel
    g_out[...] = g
    s_out[...] = s_acc.reshape(bb, NP, 1)


def kernel(h, pos, vel, g, params):
    B, N, nf = h.shape
    n_layers = len(params)
    bb = 8
    pad = NP - N

    hp = jnp.pad(h, ((0, 0), (0, pad), (0, 0)))
    posp = jnp.pad(pos, ((0, 0), (0, pad), (0, 0)))
    velp = jnp.pad(vel, ((0, 0), (0, pad), (0, 0)))
    gp = jnp.pad(g, ((0, 0), (0, pad), (0, 0)))

    i_idx = np.arange(N)[:, None]
    j_idx = np.arange(NP)[None, :]
    bmask_np = ((i_idx != j_idx) & (j_idx < N)).astype(np.float32)[:, :, None]
    bmask = jnp.asarray(bmask_np)

    st = lambda name: jnp.stack([p[name] for p in params])
    we1 = st("We1")
    wa = we1[:, :nf]
    wb = we1[:, nf:2 * nf]
    wr = we1[:, 2 * nf:]
    be1 = st("be1")[:, None, :]
    we2 = st("We2")
    be2 = st("be2")[:, None, :]
    wc1 = st("Wc1")
    bc1 = st("bc1")[:, None, :]
    wc2 = st("Wc2")
    wn1 = st("Wn1")
    wn1h = wn1[:, :nf]
    wn1a = wn1[:, nf:]
    bn1 = st("bn1")[:, None, :]
    wn2 = st("Wn2")
    bn2 = st("bn2")[:, None, :]
    ws = st("Ws")
    bs = st("bs")[:, :, None]

    def wspec(x):
        return pl.BlockSpec(x.shape, lambda i: (0,) * x.ndim)

    def bspec(last):
        return pl.BlockSpec((bb, NP, last), lambda i: (i, 0, 0))

    weights = (wa, wb, wr, be1, we2, be2, wc1, bc1, wc2,
               wn1h, wn1a, bn1, wn2, bn2, ws, bs)

    outs = pl.pallas_call(
        functools.partial(_enflow_kernel, n_layers, N, bb),
        grid=(B // bb,),
        in_specs=[bspec(nf), bspec(3), bspec(3), bspec(nf), wspec(bmask)]
                 + [wspec(w) for w in weights],
        out_specs=[bspec(nf), bspec(3), bspec(3), bspec(nf), bspec(1)],
        out_shape=[
            jax.ShapeDtypeStruct((B, NP, nf), jnp.float32),
            jax.ShapeDtypeStruct((B, NP, 3), jnp.float32),
            jax.ShapeDtypeStruct((B, NP, 3), jnp.float32),
            jax.ShapeDtypeStruct((B, NP, nf), jnp.float32),
            jax.ShapeDtypeStruct((B, NP, 1), jnp.float32),
        ],
        compiler_params=pltpu.CompilerParams(
            dimension_semantics=("parallel",)),
    )(hp, posp, velp, gp, bmask, *weights)

    h_o, pos_o, vel_o, g_o, s_o = outs
    ldj = jnp.sum(s_o[:, :N])
    return (h_o[:, :N], pos_o[:, :N], vel_o[:, :N], g_o[:, :N], ldj)
